# baseline (speedup 1.0000x reference)
import functools

import jax
import jax.numpy as jnp
from jax import lax
from jax.experimental import pallas as pl
from jax.experimental.pallas import tpu as pltpu
from jax.experimental.pallas import tpu_sc as plsc

_ALPHA = 0.7
_BETA = 0.3
_SMOOTH = 1.0
_TOP_PERCENT = 0.05
_TAU = 0.1
_BCE_WEIGHT = 0.5
_FOCAL_WEIGHT = 0.5

_N = 65536
_K = max(1, int(_TOP_PERCENT * _N))
_NS = 16
_PER = _N // _NS
_CH = _PER // 16

_MASK31 = 0x7FFFFFFF


def _sc_select_body(logits_hbm, out_hbm, x_v, keys_v, cand_v, hist_v, all_v,
                    m_v, sem, sh0, sh1, sh2, sh3):
    shared = [sh0, sh1, sh2, sh3]
    cid = lax.axis_index("c")
    sid = lax.axis_index("s")
    cp = pltpu.async_copy(logits_hbm.at[pl.ds(sid * _PER, _PER)], x_v, sem)

    ones = jnp.ones((16,), jnp.int32)
    zeros16 = jnp.zeros((16,), jnp.int32)
    iota16 = lax.broadcasted_iota(jnp.int32, (16,), 0)

    for j in range(16):
        hist_v[pl.ds(j * 16, 16)] = zeros16

    cp.wait()

    def _select(kp):
        carry = jnp.int32(0)
        lt_acc = zeros16
        below_acc = zeros16
        for c in range(15, -1, -1):
            h = lax.rev(m_v[c, pl.ds(0, 16)], (0,))
            s_run = plsc.cumsum(h) + carry
            lt = s_run < kp
            lt_acc = lt_acc + jnp.where(lt, 1, 0)
            below_acc = jnp.maximum(below_acc, jnp.where(lt, s_run, 0))
            carry = jnp.max(s_run)
        b = 255 - jnp.sum(lt_acc)
        return b, kp - jnp.max(below_acc)

    def _merge_and_select(p, kp):
        pltpu.sync_copy(hist_v, shared[p].at[sid])
        plsc.subcore_barrier()
        pltpu.sync_copy(shared[p], all_v)
        for c in range(16):
            acc = all_v[0, pl.ds(c * 16, 16)]
            for r in range(1, 16):
                acc = acc + all_v[r, pl.ds(c * 16, 16)]
            m_v[c, pl.ds(0, 16)] = acc
        b, kp = _select(kp)
        for j in range(16):
            hist_v[pl.ds(j * 16, 16)] = zeros16
        return b, kp

    kp = jnp.int32(_K)

    def _h0(i, c):
        for u in range(4):
            off = (i * 4 + u) * 16
            bits = plsc.bitcast(x_v[pl.ds(off, 16)], jnp.int32)
            kv = bits ^ ((bits >> 31) & _MASK31)
            keys_v[pl.ds(off, 16)] = kv
            dig = (kv >> 24) + 128
            plsc.addupdate_scatter(hist_v, [dig], ones)
        return c

    lax.fori_loop(0, _CH // 4, _h0, 0)
    b0, kp = _merge_and_select(0, kp)
    pref = b0 - 128

    sh16 = jnp.full((16,), 16, jnp.int32)

    def _h1(i, off, _pref=pref):
        kv = keys_v[pl.ds(i * 16, 16)]
        msk = (kv >> 24) == _pref
        dig = lax.shift_right_logical(kv, sh16) & 255
        plsc.addupdate_scatter(hist_v, [dig], ones, mask=msk)
        pos = plsc.cumsum(jnp.where(msk, 1, 0)) - 1 + off
        plsc.store_scatter(cand_v, [pos], kv, mask=msk)
        return off + plsc.all_reduce_population_count(msk)

    off = lax.fori_loop(0, _CH, _h1, zeros16, unroll=4)
    b1, kp = _merge_and_select(1, kp)
    pref = (pref << 8) | b1

    cnt = jnp.max(off)
    nch = (cnt + 15) >> 4

    sh8 = jnp.full((16,), 8, jnp.int32)

    def _h2(i, c, _pref=pref):
        kv = cand_v[pl.ds(i * 16, 16)]
        valid = iota16 < (off - i * 16)
        msk = valid & ((kv >> 16) == _pref)
        dig = lax.shift_right_logical(kv, sh8) & 255
        plsc.addupdate_scatter(hist_v, [dig], ones, mask=msk)
        return c

    lax.fori_loop(0, nch, _h2, 0)
    b2, kp = _merge_and_select(2, kp)
    pref = (pref << 8) | b2

    def _h3(i, c, _pref=pref):
        kv = cand_v[pl.ds(i * 16, 16)]
        valid = iota16 < (off - i * 16)
        msk = valid & ((kv >> 8) == _pref)
        dig = kv & 255
        plsc.addupdate_scatter(hist_v, [dig], ones, mask=msk)
        return c

    lax.fori_loop(0, nch, _h3, 0)
    b3, _ = _merge_and_select(3, kp)
    pref = (pref << 8) | b3

    fb = jnp.where(pref >= 0, pref, pref ^ _MASK31)
    keys_v[pl.ds(0, 16)] = jnp.broadcast_to(fb, (16,))

    @pl.when(jnp.logical_and(cid == 0, sid == 0))
    def _():
        pltpu.sync_copy(keys_v.at[pl.ds(0, 16)], out_hbm)


@functools.cache
def _sc_select():
    return pl.kernel(
        _sc_select_body,
        out_type=jax.ShapeDtypeStruct((16,), jnp.int32),
        mesh=plsc.VectorSubcoreMesh(core_axis_name="c", subcore_axis_name="s",
                                    num_cores=1, num_subcores=_NS),
        compiler_params=pltpu.CompilerParams(needs_layout_passes=False),
        scratch_types=[
            pltpu.VMEM((_PER,), jnp.float32),
            pltpu.VMEM((_PER,), jnp.int32),
            pltpu.VMEM((_PER + 16,), jnp.int32),
            pltpu.VMEM((256,), jnp.int32),
            pltpu.VMEM((16, 256), jnp.int32),
            pltpu.VMEM((16, 16), jnp.int32),
            pltpu.SemaphoreType.DMA,
            pltpu.VMEM_SHARED((16, 256), jnp.int32),
            pltpu.VMEM_SHARED((16, 256), jnp.int32),
            pltpu.VMEM_SHARED((16, 256), jnp.int32),
            pltpu.VMEM_SHARED((16, 256), jnp.int32),
        ],
    )


_GRID = 8
_ROWS = 512 // _GRID


def _tc_pre_body(x_ref, t_ref, s_ref, e_ref, sw_ref, sums_ref, acc):
    i = pl.program_id(0)

    @pl.when(i == 0)
    def _():
        for j in range(4):
            acc[j] = 0.0

    x = x_ref[...]
    t = t_ref[...].astype(jnp.float32)
    s = s_ref[...]
    w = jnp.where(s == 0, 2.0,
                  jnp.where(s == 1, 1.0,
                            jnp.where(s == 2, 0.5, 1.5))).astype(jnp.float32)
    eabs = jnp.exp(-jnp.abs(x))
    inv = 1.0 / (1.0 + eabs)
    probs = jnp.where(x >= 0, inv, eabs * inv)
    e_ref[...] = jnp.exp(probs * (-1.0 / _TAU))
    sw_ref[...] = w * (2.0 * t - 1.0)

    bce = jnp.maximum(x, 0.0) - x * t + jnp.log(1.0 + eabs)
    pt = jnp.where(t == 1.0, probs, 1.0 - probs)

    acc[0] += jnp.sum(w * t)
    acc[1] += jnp.sum(w)
    acc[2] += jnp.sum(w * bce)
    acc[3] += jnp.sum(w * (1.0 - pt) * (1.0 - pt) * bce)

    @pl.when(i == _GRID - 1)
    def _():
        for j in range(4):
            sums_ref[j] = acc[j]


_tc_pre = pl.pallas_call(
    _tc_pre_body,
    grid=(_GRID,),
    out_shape=(
        jax.ShapeDtypeStruct((512, 128), jnp.float32),
        jax.ShapeDtypeStruct((512, 128), jnp.float32),
        jax.ShapeDtypeStruct((4,), jnp.float32),
    ),
    in_specs=[
        pl.BlockSpec((_ROWS, 128), lambda i: (i, 0)),
        pl.BlockSpec((_ROWS, 128), lambda i: (i, 0)),
        pl.BlockSpec((_ROWS, 128), lambda i: (i, 0)),
    ],
    out_specs=(
        pl.BlockSpec((_ROWS, 128), lambda i: (i, 0)),
        pl.BlockSpec((_ROWS, 128), lambda i: (i, 0)),
        pl.BlockSpec(memory_space=pltpu.SMEM),
    ),
    scratch_shapes=[pltpu.SMEM((4,), jnp.float32)],
)


def _tc_fin_body(e_ref, sw_ref, q_ref, sums_ref, o_ref, acc):
    i = pl.program_id(0)

    @pl.when(i == 0)
    def _():
        acc[0] = 0.0
        acc[1] = 0.0

    q_logit = lax.bitcast_convert_type(q_ref[0], jnp.float32)
    q = 1.0 / (1.0 + jnp.exp(-q_logit))
    c = jnp.exp(q / _TAU)
    m = 1.0 / (1.0 + e_ref[...] * c)
    sw = sw_ref[...]
    acc[0] += jnp.sum(m * jnp.maximum(sw, 0.0))
    acc[1] += jnp.sum(m * jnp.maximum(-sw, 0.0))

    @pl.when(i == _GRID - 1)
    def _():
        tp = acc[0]
        fp = acc[1]
        wt_s = sums_ref[0]
        ws = sums_ref[1]
        bce_s = sums_ref[2]
        foc_s = sums_ref[3]
        fn = wt_s - tp
        tversky = (tp + _SMOOTH) / (tp + _ALPHA * fp + _BETA * fn + _SMOOTH)
        o_ref[0, 0] = (1.0 - tversky
                       + (_BCE_WEIGHT * bce_s + _FOCAL_WEIGHT * foc_s)
                       / (ws + 1e-12))


_tc_fin = pl.pallas_call(
    _tc_fin_body,
    grid=(_GRID,),
    out_shape=jax.ShapeDtypeStruct((1, 1), jnp.float32),
    in_specs=[
        pl.BlockSpec((_ROWS, 128), lambda i: (i, 0)),
        pl.BlockSpec((_ROWS, 128), lambda i: (i, 0)),
        pl.BlockSpec(memory_space=pltpu.SMEM),
        pl.BlockSpec(memory_space=pltpu.SMEM),
    ],
    out_specs=pl.BlockSpec(memory_space=pltpu.SMEM),
    scratch_shapes=[pltpu.SMEM((2,), jnp.float32)],
)


@jax.jit
def kernel(logits, targets, sources):
    qbits = _sc_select()(logits)
    x2 = logits.reshape(512, 128)
    t2 = targets.reshape(512, 128)
    s2 = sources.reshape(512, 128)
    e2, sw2, sums = _tc_pre(x2, t2, s2)
    out = _tc_fin(e2, sw2, qbits, sums)
    return out.reshape(())


# speedup vs baseline: 1.1078x; 1.1078x over previous
import functools

import jax
import jax.numpy as jnp
from jax import lax
from jax.experimental import pallas as pl
from jax.experimental.pallas import tpu as pltpu
from jax.experimental.pallas import tpu_sc as plsc

_ALPHA = 0.7
_BETA = 0.3
_SMOOTH = 1.0
_TOP_PERCENT = 0.05
_TAU = 0.1
_BCE_WEIGHT = 0.5
_FOCAL_WEIGHT = 0.5

_N = 65536
_K = max(1, int(_TOP_PERCENT * _N))
_NS = 16
_PER = _N // _NS
_CH = _PER // 16

_MASK31 = 0x7FFFFFFF


def _sc_select_body(logits_hbm, out_hbm, x_v, keys_v, cand_v, hist_v, all_v,
                    m_v, sem, sh0, sh1, sh2, sh3):
    shared = [sh0, sh1, sh2, sh3]
    cid = lax.axis_index("c")
    sid = lax.axis_index("s")
    cp = pltpu.async_copy(logits_hbm.at[pl.ds(sid * _PER, _PER)], x_v, sem)

    ones = jnp.ones((16,), jnp.int32)
    zeros16 = jnp.zeros((16,), jnp.int32)
    iota16 = lax.broadcasted_iota(jnp.int32, (16,), 0)

    for j in range(16):
        hist_v[pl.ds(j * 16, 16)] = zeros16

    cp.wait()

    def _select(kp):
        carry = jnp.int32(0)
        lt_acc = zeros16
        below_acc = zeros16
        for c in range(15, -1, -1):
            h = lax.rev(m_v[c, pl.ds(0, 16)], (0,))
            s_run = plsc.cumsum(h) + carry
            lt = s_run < kp
            lt_acc = lt_acc + jnp.where(lt, 1, 0)
            below_acc = jnp.maximum(below_acc, jnp.where(lt, s_run, 0))
            carry = jnp.max(s_run)
        b = 255 - jnp.sum(lt_acc)
        return b, kp - jnp.max(below_acc)

    def _merge_and_select(p, kp):
        pltpu.sync_copy(hist_v, shared[p].at[sid])
        plsc.subcore_barrier()
        pltpu.sync_copy(shared[p], all_v)
        for c in range(16):
            acc = all_v[0, pl.ds(c * 16, 16)]
            for r in range(1, 16):
                acc = acc + all_v[r, pl.ds(c * 16, 16)]
            m_v[c, pl.ds(0, 16)] = acc
        b, kp = _select(kp)
        for j in range(16):
            hist_v[pl.ds(j * 16, 16)] = zeros16
        return b, kp

    kp = jnp.int32(_K)

    def _h0(i, c):
        for u in range(4):
            off = (i * 4 + u) * 16
            bits = plsc.bitcast(x_v[pl.ds(off, 16)], jnp.int32)
            kv = bits ^ ((bits >> 31) & _MASK31)
            keys_v[pl.ds(off, 16)] = kv
            dig = (kv >> 24) + 128
            plsc.addupdate_scatter(hist_v, [dig], ones)
        return c

    lax.fori_loop(0, _CH // 4, _h0, 0)
    b0, kp = _merge_and_select(0, kp)
    pref = b0 - 128

    sh16 = jnp.full((16,), 16, jnp.int32)

    def _h1(i, off, _pref=pref):
        kv = keys_v[pl.ds(i * 16, 16)]
        msk = (kv >> 24) == _pref
        dig = lax.shift_right_logical(kv, sh16) & 255
        plsc.addupdate_scatter(hist_v, [dig], ones, mask=msk)
        pos = plsc.cumsum(jnp.where(msk, 1, 0)) - 1 + off
        plsc.store_scatter(cand_v, [pos], kv, mask=msk)
        return off + plsc.all_reduce_population_count(msk)

    off = lax.fori_loop(0, _CH, _h1, zeros16, unroll=4)
    b1, kp = _merge_and_select(1, kp)
    pref = (pref << 8) | b1

    cnt = jnp.max(off)
    nch = (cnt + 15) >> 4

    sh8 = jnp.full((16,), 8, jnp.int32)

    def _h2(i, c, _pref=pref):
        kv = cand_v[pl.ds(i * 16, 16)]
        valid = iota16 < (off - i * 16)
        msk = valid & ((kv >> 16) == _pref)
        dig = lax.shift_right_logical(kv, sh8) & 255
        plsc.addupdate_scatter(hist_v, [dig], ones, mask=msk)
        return c

    lax.fori_loop(0, nch, _h2, 0)
    b2, kp = _merge_and_select(2, kp)
    pref = (pref << 8) | b2

    def _h3(i, c, _pref=pref):
        kv = cand_v[pl.ds(i * 16, 16)]
        valid = iota16 < (off - i * 16)
        msk = valid & ((kv >> 8) == _pref)
        dig = kv & 255
        plsc.addupdate_scatter(hist_v, [dig], ones, mask=msk)
        return c

    lax.fori_loop(0, nch, _h3, 0)
    b3, _ = _merge_and_select(3, kp)
    pref = (pref << 8) | b3

    fb = jnp.where(pref >= 0, pref, pref ^ _MASK31)
    keys_v[pl.ds(0, 16)] = jnp.broadcast_to(fb, (16,))

    @pl.when(jnp.logical_and(cid == 0, sid == 0))
    def _():
        pltpu.sync_copy(keys_v.at[pl.ds(0, 16)], out_hbm)


@functools.cache
def _sc_select():
    return pl.kernel(
        _sc_select_body,
        out_type=jax.ShapeDtypeStruct((16,), jnp.int32),
        mesh=plsc.VectorSubcoreMesh(core_axis_name="c", subcore_axis_name="s",
                                    num_cores=1, num_subcores=_NS),
        compiler_params=pltpu.CompilerParams(needs_layout_passes=False),
        scratch_types=[
            pltpu.VMEM((_PER,), jnp.float32),
            pltpu.VMEM((_PER,), jnp.int32),
            pltpu.VMEM((_PER + 16,), jnp.int32),
            pltpu.VMEM((256,), jnp.int32),
            pltpu.VMEM((16, 256), jnp.int32),
            pltpu.VMEM((16, 16), jnp.int32),
            pltpu.SemaphoreType.DMA,
            pltpu.VMEM_SHARED((16, 256), jnp.int32),
            pltpu.VMEM_SHARED((16, 256), jnp.int32),
            pltpu.VMEM_SHARED((16, 256), jnp.int32),
            pltpu.VMEM_SHARED((16, 256), jnp.int32),
        ],
    )


_GRID = 8
_ROWS = 512 // _GRID


def _tc_pre_body(x_ref, t_ref, s_ref, e_ref, sw_ref, sums_ref, acc):
    i = pl.program_id(0)

    @pl.when(i == 0)
    def _():
        for j in range(4):
            acc[j] = 0.0

    x = x_ref[...]
    t = t_ref[...].astype(jnp.float32)
    s = s_ref[...]
    w = jnp.where(s == 0, 2.0,
                  jnp.where(s == 1, 1.0,
                            jnp.where(s == 2, 0.5, 1.5))).astype(jnp.float32)
    eabs = jnp.exp(-jnp.abs(x))
    inv = 1.0 / (1.0 + eabs)
    probs = jnp.where(x >= 0, inv, eabs * inv)
    e_ref[...] = jnp.exp(probs * (-1.0 / _TAU))
    sw_ref[...] = w * (2.0 * t - 1.0)

    bce = jnp.maximum(x, 0.0) - x * t + jnp.log(1.0 + eabs)
    pt = jnp.where(t == 1.0, probs, 1.0 - probs)

    acc[0] += jnp.sum(w * t)
    acc[1] += jnp.sum(w)
    acc[2] += jnp.sum(w * bce)
    acc[3] += jnp.sum(w * (1.0 - pt) * (1.0 - pt) * bce)

    @pl.when(i == _GRID - 1)
    def _():
        for j in range(4):
            sums_ref[j] = acc[j]


_tc_pre = pl.pallas_call(
    _tc_pre_body,
    grid=(_GRID,),
    out_shape=(
        jax.ShapeDtypeStruct((512, 128), jnp.float32),
        jax.ShapeDtypeStruct((512, 128), jnp.float32),
        jax.ShapeDtypeStruct((4,), jnp.float32),
    ),
    in_specs=[
        pl.BlockSpec((_ROWS, 128), lambda i: (i, 0)),
        pl.BlockSpec((_ROWS, 128), lambda i: (i, 0)),
        pl.BlockSpec((_ROWS, 128), lambda i: (i, 0)),
    ],
    out_specs=(
        pl.BlockSpec((_ROWS, 128), lambda i: (i, 0)),
        pl.BlockSpec((_ROWS, 128), lambda i: (i, 0)),
        pl.BlockSpec(memory_space=pltpu.SMEM),
    ),
    scratch_shapes=[pltpu.SMEM((4,), jnp.float32)],
)


def _tc_fin_body(e_ref, sw_ref, q_ref, sums_ref, o_ref):
    q_logit = lax.bitcast_convert_type(q_ref[0], jnp.float32)
    q = 1.0 / (1.0 + jnp.exp(-q_logit))
    c = jnp.exp(q / _TAU)
    m = 1.0 / (1.0 + e_ref[...] * c)
    sw = sw_ref[...]
    tp = jnp.sum(m * jnp.maximum(sw, 0.0))
    fp = jnp.sum(m * jnp.maximum(-sw, 0.0))
    wt_s = sums_ref[0]
    ws = sums_ref[1]
    bce_s = sums_ref[2]
    foc_s = sums_ref[3]
    fn = wt_s - tp
    tversky = (tp + _SMOOTH) / (tp + _ALPHA * fp + _BETA * fn + _SMOOTH)
    o_ref[0, 0] = (1.0 - tversky
                   + (_BCE_WEIGHT * bce_s + _FOCAL_WEIGHT * foc_s)
                   / (ws + 1e-12))


_tc_fin = pl.pallas_call(
    _tc_fin_body,
    out_shape=jax.ShapeDtypeStruct((1, 1), jnp.float32),
    in_specs=[
        pl.BlockSpec((512, 128), lambda: (0, 0)),
        pl.BlockSpec((512, 128), lambda: (0, 0)),
        pl.BlockSpec(memory_space=pltpu.SMEM),
        pl.BlockSpec(memory_space=pltpu.SMEM),
    ],
    out_specs=pl.BlockSpec(memory_space=pltpu.SMEM),
)


@jax.jit
def kernel(logits, targets, sources):
    qbits = _sc_select()(logits)
    x2 = logits.reshape(512, 128)
    t2 = targets.reshape(512, 128)
    s2 = sources.reshape(512, 128)
    e2, sw2, sums = _tc_pre(x2, t2, s2)
    out = _tc_fin(e2, sw2, qbits, sums)
    return out.reshape(())
